# baseline (speedup 1.0000x reference)
import jax
import jax.numpy as jnp
import numpy as np
from jax.experimental import pallas as pl

B = 4096
D = 128
H = 4
NN = 4
NSPEC = 3

_L = np.arange(NN * NN * H)
_LI, _LJ, _LK = _L // 16, (_L // 4) % 4, _L % 4

_T = np.zeros((NN, 2 * H, 3 * 64), dtype=np.float32)
for n in range(NN):
    for l in range(64):
        _T[n, _LK[l], l] = 1.0 if _LJ[l] == n else 0.0
        _T[n, H + _LK[l], 64 + l] = 1.0 if _LI[l] == n else 0.0
        _T[n, _LK[l], 128 + l] = 1.0
_T = _T.reshape(NN * 2 * H, 3 * 64)

_S = ((_LK[:, None] == _LK[None, :]) & (_LI[:, None] == _LI[None, :])
      ).astype(np.float32)

_SP = (np.arange(16 * D)[None, :] // D == np.arange(16)[:, None]
       ).astype(np.float32) * (1.0 / H)


def _gat_body(spec_ref, shared_ref, wt_ref, ws_ref, a8_ref, t_ref, s_ref,
              sp_ref, bias_ref, xo_ref, fused_ref):
    bb = spec_ref.shape[1]
    spec2d = spec_ref[...].reshape(NSPEC * bb, D)
    sh2d = shared_ref[0]

    wa = jnp.dot(wt_ref[...], a8_ref[...], preferred_element_type=jnp.float32)
    p_spec = jnp.dot(spec2d, wa, preferred_element_type=jnp.float32)
    p = [p_spec[n * bb:(n + 1) * bb, :] for n in range(NSPEC)]
    p.append(jnp.dot(sh2d, wa, preferred_element_type=jnp.float32))

    q = [jnp.dot(p[n], t_ref[2 * H * n:2 * H * (n + 1), :],
                 preferred_element_type=jnp.float32) for n in range(NN)]
    as_t = q[0][:, 0:64] + q[1][:, 0:64] + q[2][:, 0:64] + q[3][:, 0:64]
    ad_r = q[0][:, 64:128] + q[1][:, 64:128] + q[2][:, 64:128] + q[3][:, 64:128]
    ms = jnp.maximum(jnp.maximum(q[0][:, 128:192], q[1][:, 128:192]),
                     jnp.maximum(q[2][:, 128:192], q[3][:, 128:192]))

    def leaky(v):
        return jnp.where(v > 0, v, 0.2 * v)

    lg = leaky(as_t + ad_r)
    m = leaky(ms + ad_r)
    e = jnp.exp(lg - m)
    s = jnp.dot(e, s_ref[...], preferred_element_type=jnp.float32)
    c = e * (1.0 / jnp.maximum(s, 1e-16))

    cb16 = c.astype(jnp.bfloat16)
    sp = sp_ref[...].astype(jnp.bfloat16)
    ws = ws_ref[...].astype(jnp.bfloat16)
    xb = [spec2d[n * bb:(n + 1) * bb, :].astype(jnp.bfloat16)
          for n in range(NSPEC)] + [sh2d.astype(jnp.bfloat16)]
    bias = bias_ref[...]

    acc_fused = None
    for i in range(NN):
        cb = jnp.dot(cb16[:, i * 16:(i + 1) * 16], sp,
                     preferred_element_type=jnp.float32
                     ).astype(jnp.bfloat16)
        mixes = []
        for k in range(H):
            mk = None
            for j in range(NN):
                blk = (j * H + k) * D
                t = cb[:, blk:blk + D] * xb[j]
                mk = t if mk is None else mk + t
            mixes.append(mk)
        mix = jnp.concatenate(mixes, axis=1)
        merged = jnp.dot(mix, ws, preferred_element_type=jnp.float32) + bias
        xi = jnp.where(merged > 0, merged, jnp.exp(merged) - 1.0)
        xo_ref[:, i, :] = xi
        acc_fused = xi if acc_fused is None else acc_fused + xi
    fused_ref[...] = acc_fused * (1.0 / NN)


def kernel(specific_features, shared_features, W, att_src, att_dst, bias):
    wt = W.T
    ws = W.reshape(H, D, D).swapaxes(1, 2).reshape(H * D, D)
    bias2 = bias.reshape(1, D)
    eye = jnp.asarray(np.eye(H, dtype=np.float32))
    a_src_blk = (att_src[:, :, None] * eye[:, None, :]).reshape(H * D, H)
    a_dst_blk = (att_dst[:, :, None] * eye[:, None, :]).reshape(H * D, H)
    a8 = jnp.concatenate([a_src_blk, a_dst_blk], axis=1)

    tmat = jnp.asarray(_T)
    smat = jnp.asarray(_S)
    spmat = jnp.asarray(_SP)

    bb = 1024
    grid = (B // bb,)
    xo, fused = pl.pallas_call(
        _gat_body,
        grid=grid,
        in_specs=[
            pl.BlockSpec((NSPEC, bb, D), lambda i: (0, i, 0)),
            pl.BlockSpec((1, bb, D), lambda i: (0, i, 0)),
            pl.BlockSpec((D, H * D), lambda i: (0, 0)),
            pl.BlockSpec((H * D, D), lambda i: (0, 0)),
            pl.BlockSpec((H * D, 2 * H), lambda i: (0, 0)),
            pl.BlockSpec((NN * 2 * H, 3 * 64), lambda i: (0, 0)),
            pl.BlockSpec((64, 64), lambda i: (0, 0)),
            pl.BlockSpec((16, 16 * D), lambda i: (0, 0)),
            pl.BlockSpec((1, D), lambda i: (0, 0)),
        ],
        out_specs=[
            pl.BlockSpec((bb, NN, D), lambda i: (i, 0, 0)),
            pl.BlockSpec((bb, D), lambda i: (i, 0)),
        ],
        out_shape=[
            jax.ShapeDtypeStruct((B, NN, D), jnp.float32),
            jax.ShapeDtypeStruct((B, D), jnp.float32),
        ],
    )(specific_features, shared_features, wt, ws, a8, tmat, smat, spmat,
      bias2)
    return fused, xo


# speedup vs baseline: 1.1265x; 1.1265x over previous
import jax
import jax.numpy as jnp
import numpy as np
from jax.experimental import pallas as pl

B = 4096
D = 128
H = 4
NN = 4
NSPEC = 3

_L = np.arange(NN * NN * H)
_LI, _LJ, _LK = _L // 16, (_L // 4) % 4, _L % 4

_T = np.zeros((NN, 2 * H, 3 * 64), dtype=np.float32)
for n in range(NN):
    for l in range(64):
        _T[n, _LK[l], l] = 1.0 if _LJ[l] == n else 0.0
        _T[n, H + _LK[l], 64 + l] = 1.0 if _LI[l] == n else 0.0
        _T[n, _LK[l], 128 + l] = 1.0
_T = _T.reshape(NN * 2 * H, 3 * 64)

_S = ((_LK[:, None] == _LK[None, :]) & (_LI[:, None] == _LI[None, :])
      ).astype(np.float32)

_SP = (np.arange(16 * D)[None, :] // D == np.arange(16)[:, None]
       ).astype(np.float32) * (1.0 / H)


_DN_T = (((1,), (1,)), ((), ()))


def _gat_body(spec_ref, shared_ref, w_ref, asrc_ref, adst_ref, t_ref, s_ref,
              sp_ref, bias_ref, xo_ref, fused_ref):
    bb = spec_ref.shape[1]
    spec2d = spec_ref[...].reshape(NSPEC * bb, D)
    sh2d = shared_ref[0]

    w = w_ref[...]
    wk = [w[k * D:(k + 1) * D, :] for k in range(H)]
    rows = ([jnp.dot(asrc_ref[k:k + 1, :], wk[k],
                     preferred_element_type=jnp.float32) for k in range(H)]
            + [jnp.dot(adst_ref[k:k + 1, :], wk[k],
                       preferred_element_type=jnp.float32) for k in range(H)])
    pat = jnp.concatenate(rows, axis=0)
    p_spec = jax.lax.dot_general(spec2d, pat, _DN_T,
                                 preferred_element_type=jnp.float32)
    p = [p_spec[n * bb:(n + 1) * bb, :] for n in range(NSPEC)]
    p.append(jax.lax.dot_general(sh2d, pat, _DN_T,
                                 preferred_element_type=jnp.float32))

    q = [jnp.dot(p[n], t_ref[2 * H * n:2 * H * (n + 1), :],
                 preferred_element_type=jnp.float32) for n in range(NN)]
    as_t = q[0][:, 0:64] + q[1][:, 0:64] + q[2][:, 0:64] + q[3][:, 0:64]
    ad_r = q[0][:, 64:128] + q[1][:, 64:128] + q[2][:, 64:128] + q[3][:, 64:128]
    ms = jnp.maximum(jnp.maximum(q[0][:, 128:192], q[1][:, 128:192]),
                     jnp.maximum(q[2][:, 128:192], q[3][:, 128:192]))

    def leaky(v):
        return jnp.where(v > 0, v, 0.2 * v)

    lg = leaky(as_t + ad_r)
    m = leaky(ms + ad_r)
    e = jnp.exp(lg - m)
    s = jnp.dot(e, s_ref[...], preferred_element_type=jnp.float32)
    c = e * (1.0 / jnp.maximum(s, 1e-16))

    cb16 = c.astype(jnp.bfloat16)
    sp = sp_ref[...].astype(jnp.bfloat16)
    wkb = [wk[k].astype(jnp.bfloat16) for k in range(H)]
    xb = [spec2d[n * bb:(n + 1) * bb, :].astype(jnp.bfloat16)
          for n in range(NSPEC)] + [sh2d.astype(jnp.bfloat16)]
    bias = bias_ref[...]

    acc_fused = None
    for i in range(NN):
        cb = jnp.dot(cb16[:, i * 16:(i + 1) * 16], sp,
                     preferred_element_type=jnp.float32
                     ).astype(jnp.bfloat16)
        merged = None
        for k in range(H):
            mk = None
            for j in range(NN):
                blk = (j * H + k) * D
                t = cb[:, blk:blk + D] * xb[j]
                mk = t if mk is None else mk + t
            mg = jax.lax.dot_general(mk, wkb[k], _DN_T,
                                     preferred_element_type=jnp.float32)
            merged = mg if merged is None else merged + mg
        merged = merged + bias
        xi = jnp.where(merged > 0, merged, jnp.exp(merged) - 1.0)
        xo_ref[:, i, :] = xi
        acc_fused = xi if acc_fused is None else acc_fused + xi
    fused_ref[...] = acc_fused * (1.0 / NN)


def kernel(specific_features, shared_features, W, att_src, att_dst, bias):
    bias2 = bias.reshape(1, D)
    tmat = jnp.asarray(_T)
    smat = jnp.asarray(_S)
    spmat = jnp.asarray(_SP)

    bb = 1024
    grid = (B // bb,)
    xo, fused = pl.pallas_call(
        _gat_body,
        grid=grid,
        in_specs=[
            pl.BlockSpec((NSPEC, bb, D), lambda i: (0, i, 0)),
            pl.BlockSpec((1, bb, D), lambda i: (0, i, 0)),
            pl.BlockSpec((H * D, D), lambda i: (0, 0)),
            pl.BlockSpec((H, D), lambda i: (0, 0)),
            pl.BlockSpec((H, D), lambda i: (0, 0)),
            pl.BlockSpec((NN * 2 * H, 3 * 64), lambda i: (0, 0)),
            pl.BlockSpec((64, 64), lambda i: (0, 0)),
            pl.BlockSpec((16, 16 * D), lambda i: (0, 0)),
            pl.BlockSpec((1, D), lambda i: (0, 0)),
        ],
        out_specs=[
            pl.BlockSpec((bb, NN, D), lambda i: (i, 0, 0)),
            pl.BlockSpec((bb, D), lambda i: (i, 0)),
        ],
        out_shape=[
            jax.ShapeDtypeStruct((B, NN, D), jnp.float32),
            jax.ShapeDtypeStruct((B, D), jnp.float32),
        ],
    )(specific_features, shared_features, W, att_src, att_dst, tmat, smat,
      spmat, bias2)
    return fused, xo
